# baseline (speedup 1.0000x reference)
import jax
import jax.numpy as jnp
from jax.experimental import pallas as pl

_ROWS_PER_BLOCK = 128


def _fused_body(word_ref, age_ref, w_ref, b_ref, tab_ref, out_ref):
    R, S, D = word_ref.shape
    E = w_ref.shape[1]
    EA = tab_ref.shape[1]
    w2 = word_ref[...].reshape(R * S, D)
    lin = jnp.dot(w2, w_ref[...], preferred_element_type=jnp.float32) + b_ref[...]
    idx3 = age_ref[...][..., None]
    cols3 = jax.lax.broadcasted_iota(jnp.int32, (1, 1, tab_ref.shape[0]), 2)
    onehot = (idx3 == cols3).astype(jnp.float32).reshape(R * S, tab_ref.shape[0])
    emb = jnp.dot(onehot, tab_ref[...], preferred_element_type=jnp.float32)
    out = jnp.concatenate((lin, emb), axis=-1)
    out_ref[...] = out.reshape(R, S, E + EA)


def kernel(word, age, age_table, W, b):
    B, S, D = word.shape
    E = W.shape[1]
    A, EA = age_table.shape
    age32 = jnp.asarray(age, jnp.int32)
    AP = 128
    tab = jnp.zeros((AP, EA), jnp.float32).at[:A].set(age_table)
    b2 = b.reshape(1, E)

    R = _ROWS_PER_BLOCK
    grid = (B // R,)
    out = pl.pallas_call(
        _fused_body,
        grid=grid,
        in_specs=[
            pl.BlockSpec((R, S, D), lambda i: (i, 0, 0)),
            pl.BlockSpec((R, S), lambda i: (i, 0)),
            pl.BlockSpec((D, E), lambda i: (0, 0)),
            pl.BlockSpec((1, E), lambda i: (0, 0)),
            pl.BlockSpec((AP, EA), lambda i: (0, 0)),
        ],
        out_specs=pl.BlockSpec((R, S, E + EA), lambda i: (i, 0, 0)),
        out_shape=jax.ShapeDtypeStruct((B, S, E + EA), jnp.float32),
    )(word, age32, W, b2, tab)
    return out


# speedup vs baseline: 3.8105x; 3.8105x over previous
import jax
import jax.numpy as jnp
from jax.experimental import pallas as pl

_LANES_PER_BLOCK = 2048


def _fused_body(word_ref, age_ref, wt_ref, b_ref, tabt_ref, out_ref):
    wblk = word_ref[0]
    lin = jnp.dot(wt_ref[...], wblk, preferred_element_type=jnp.float32) + b_ref[...]
    age_row = age_ref[0]
    rows = jax.lax.broadcasted_iota(jnp.int32, (tabt_ref.shape[1], 1), 0)
    onehot = (rows == age_row).astype(jnp.float32)
    emb = jnp.dot(tabt_ref[...], onehot, preferred_element_type=jnp.float32)
    out_ref[0] = jnp.concatenate((lin, emb), axis=0)


def kernel(word, age, age_table, W, b):
    B, S, D = word.shape
    E = W.shape[1]
    A, EA = age_table.shape

    word_t = jnp.transpose(word, (1, 2, 0))
    age_t = jnp.transpose(jnp.asarray(age, jnp.int32), (1, 0)).reshape(S, 1, B)
    W_t = W.T
    AP = 128
    tab_t = jnp.zeros((EA, AP), jnp.float32).at[:, :A].set(age_table.T)
    b_col = b.reshape(E, 1)

    CL = _LANES_PER_BLOCK
    grid = (S, B // CL)
    out_t = pl.pallas_call(
        _fused_body,
        grid=grid,
        in_specs=[
            pl.BlockSpec((1, D, CL), lambda s, j: (s, 0, j)),
            pl.BlockSpec((1, 1, CL), lambda s, j: (s, 0, j)),
            pl.BlockSpec((E, D), lambda s, j: (0, 0)),
            pl.BlockSpec((E, 1), lambda s, j: (0, 0)),
            pl.BlockSpec((EA, AP), lambda s, j: (0, 0)),
        ],
        out_specs=pl.BlockSpec((1, E + EA, CL), lambda s, j: (s, 0, j)),
        out_shape=jax.ShapeDtypeStruct((S, E + EA, B), jnp.float32),
    )(word_t, age_t, W_t, b_col, tab_t)
    return jnp.transpose(out_t, (2, 0, 1))


# speedup vs baseline: 5.1685x; 1.3564x over previous
import jax
import jax.numpy as jnp
from jax.experimental import pallas as pl

_LANES_PER_BLOCK = 4096


def _fused_body(word_ref, age_ref, wt_ref, b_ref, tabt_ref, out_ref):
    wblk = word_ref[0]
    lin = jnp.dot(wt_ref[...], wblk, preferred_element_type=jnp.float32) + b_ref[...]
    age_row = age_ref[0]
    rows = jax.lax.broadcasted_iota(jnp.int32, (tabt_ref.shape[1], 1), 0)
    onehot = (rows == age_row).astype(jnp.float32)
    emb = jnp.dot(tabt_ref[...], onehot, preferred_element_type=jnp.float32)
    out_ref[0] = jnp.concatenate((lin, emb), axis=0)


def kernel(word, age, age_table, W, b):
    B, S, D = word.shape
    E = W.shape[1]
    A, EA = age_table.shape

    word_t = jnp.transpose(word, (1, 2, 0))
    age_t = jnp.transpose(jnp.asarray(age, jnp.int32), (1, 0)).reshape(S, 1, B)
    W_t = W.T
    AP = 128
    tab_t = jnp.zeros((EA, AP), jnp.float32).at[:, :A].set(age_table.T)
    b_col = b.reshape(E, 1)

    CL = _LANES_PER_BLOCK
    grid = (S, B // CL)
    out_t = pl.pallas_call(
        _fused_body,
        grid=grid,
        in_specs=[
            pl.BlockSpec((1, D, CL), lambda s, j: (s, 0, j)),
            pl.BlockSpec((1, 1, CL), lambda s, j: (s, 0, j)),
            pl.BlockSpec((E, D), lambda s, j: (0, 0)),
            pl.BlockSpec((E, 1), lambda s, j: (0, 0)),
            pl.BlockSpec((EA, AP), lambda s, j: (0, 0)),
        ],
        out_specs=pl.BlockSpec((1, E + EA, CL), lambda s, j: (s, 0, j)),
        out_shape=jax.ShapeDtypeStruct((S, E + EA, B), jnp.float32),
    )(word_t, age_t, W_t, b_col, tab_t)
    return jnp.transpose(out_t, (2, 0, 1))


# speedup vs baseline: 6.1584x; 1.1915x over previous
import jax
import jax.numpy as jnp
from jax.experimental import pallas as pl

_LANES_PER_BLOCK = 8192


def _fused_body(word_ref, age_ref, wt_ref, b_ref, tabt_ref, out_ref):
    wblk = word_ref[0]
    lin = jnp.dot(wt_ref[...], wblk, preferred_element_type=jnp.float32) + b_ref[...]
    age_row = age_ref[0]
    rows = jax.lax.broadcasted_iota(jnp.int32, (tabt_ref.shape[1], 1), 0)
    onehot = (rows == age_row).astype(jnp.float32)
    emb = jnp.dot(tabt_ref[...], onehot, preferred_element_type=jnp.float32)
    out_ref[0] = jnp.concatenate((lin, emb), axis=0)


def kernel(word, age, age_table, W, b):
    B, S, D = word.shape
    E = W.shape[1]
    A, EA = age_table.shape

    word_t = jnp.transpose(word, (1, 2, 0))
    age_t = jnp.transpose(jnp.asarray(age, jnp.int32), (1, 0)).reshape(S, 1, B)
    W_t = W.T
    AP = 128
    tab_t = jnp.zeros((EA, AP), jnp.float32).at[:, :A].set(age_table.T)
    b_col = b.reshape(E, 1)

    CL = _LANES_PER_BLOCK
    grid = (S, B // CL)
    out_t = pl.pallas_call(
        _fused_body,
        grid=grid,
        in_specs=[
            pl.BlockSpec((1, D, CL), lambda s, j: (s, 0, j)),
            pl.BlockSpec((1, 1, CL), lambda s, j: (s, 0, j)),
            pl.BlockSpec((E, D), lambda s, j: (0, 0)),
            pl.BlockSpec((E, 1), lambda s, j: (0, 0)),
            pl.BlockSpec((EA, AP), lambda s, j: (0, 0)),
        ],
        out_specs=pl.BlockSpec((1, E + EA, CL), lambda s, j: (s, 0, j)),
        out_shape=jax.ShapeDtypeStruct((S, E + EA, B), jnp.float32),
    )(word_t, age_t, W_t, b_col, tab_t)
    return jnp.transpose(out_t, (2, 0, 1))


# speedup vs baseline: 6.4831x; 1.0527x over previous
import jax
import jax.numpy as jnp
from jax.experimental import pallas as pl

_LANES_PER_BLOCK = 16384


def _fused_body(word_ref, age_ref, wt_ref, b_ref, tabt_ref, out_ref):
    wblk = word_ref[0]
    lin = jnp.dot(wt_ref[...], wblk, preferred_element_type=jnp.float32) + b_ref[...]
    age_row = age_ref[0]
    rows = jax.lax.broadcasted_iota(jnp.int32, (tabt_ref.shape[1], 1), 0)
    onehot = (rows == age_row).astype(jnp.float32)
    emb = jnp.dot(tabt_ref[...], onehot, preferred_element_type=jnp.float32)
    out_ref[0] = jnp.concatenate((lin, emb), axis=0)


def kernel(word, age, age_table, W, b):
    B, S, D = word.shape
    E = W.shape[1]
    A, EA = age_table.shape

    word_t = jnp.transpose(word, (1, 2, 0))
    age_t = jnp.transpose(jnp.asarray(age, jnp.int32), (1, 0)).reshape(S, 1, B)
    W_t = W.T
    AP = 128
    tab_t = jnp.zeros((EA, AP), jnp.float32).at[:, :A].set(age_table.T)
    b_col = b.reshape(E, 1)

    CL = _LANES_PER_BLOCK
    grid = (S, B // CL)
    out_t = pl.pallas_call(
        _fused_body,
        grid=grid,
        in_specs=[
            pl.BlockSpec((1, D, CL), lambda s, j: (s, 0, j)),
            pl.BlockSpec((1, 1, CL), lambda s, j: (s, 0, j)),
            pl.BlockSpec((E, D), lambda s, j: (0, 0)),
            pl.BlockSpec((E, 1), lambda s, j: (0, 0)),
            pl.BlockSpec((EA, AP), lambda s, j: (0, 0)),
        ],
        out_specs=pl.BlockSpec((1, E + EA, CL), lambda s, j: (s, 0, j)),
        out_shape=jax.ShapeDtypeStruct((S, E + EA, B), jnp.float32),
    )(word_t, age_t, W_t, b_col, tab_t)
    return jnp.transpose(out_t, (2, 0, 1))


# speedup vs baseline: 6.6578x; 1.0270x over previous
import jax
import jax.numpy as jnp
from jax.experimental import pallas as pl

_LANES_PER_BLOCK = 16384


def _fused_body(word_ref, age_ref, w_ref, b_ref, tab_ref, out_ref):
    wblk = word_ref[0]
    lin = jax.lax.dot_general(
        w_ref[...], wblk, (((0,), (0,)), ((), ())),
        preferred_element_type=jnp.float32) + b_ref[...]
    age_row = age_ref[0]
    A = tab_ref.shape[0]
    rows = jax.lax.broadcasted_iota(jnp.int32, (A, 1), 0)
    onehot = (rows == age_row).astype(jnp.float32)
    emb = jax.lax.dot_general(
        tab_ref[...], onehot, (((0,), (0,)), ((), ())),
        preferred_element_type=jnp.float32)
    out_ref[0] = jnp.concatenate((lin, emb), axis=0)


def kernel(word, age, age_table, W, b):
    B, S, D = word.shape
    E = W.shape[1]
    A, EA = age_table.shape

    word_t = jnp.transpose(word, (1, 2, 0))
    age_t = jnp.transpose(jnp.asarray(age, jnp.int32), (1, 0)).reshape(S, 1, B)
    b_col = b.reshape(E, 1)

    CL = _LANES_PER_BLOCK
    grid = (S, B // CL)
    out_t = pl.pallas_call(
        _fused_body,
        grid=grid,
        in_specs=[
            pl.BlockSpec((1, D, CL), lambda s, j: (s, 0, j)),
            pl.BlockSpec((1, 1, CL), lambda s, j: (s, 0, j)),
            pl.BlockSpec((D, E), lambda s, j: (0, 0)),
            pl.BlockSpec((E, 1), lambda s, j: (0, 0)),
            pl.BlockSpec((A, EA), lambda s, j: (0, 0)),
        ],
        out_specs=pl.BlockSpec((1, E + EA, CL), lambda s, j: (s, 0, j)),
        out_shape=jax.ShapeDtypeStruct((S, E + EA, B), jnp.float32),
    )(word_t, age_t, W, b_col, age_table)
    return jnp.transpose(out_t, (2, 0, 1))


# speedup vs baseline: 6.9324x; 1.0412x over previous
import jax
import jax.numpy as jnp
from jax.experimental import pallas as pl

_LANES_PER_BLOCK = 16384


def _fused_body(word_ref, age_ref, w_ref, b_ref, tab_ref, out_ref):
    wblk = word_ref[0]
    lin = jax.lax.dot_general(
        w_ref[...], wblk, (((0,), (0,)), ((), ())),
        preferred_element_type=jnp.float32) + b_ref[...]
    age_row = age_ref[pl.ds(pl.program_id(0), 1), :]
    A = tab_ref.shape[0]
    rows = jax.lax.broadcasted_iota(jnp.int32, (A, 1), 0)
    onehot = (rows == age_row).astype(jnp.float32)
    emb = jax.lax.dot_general(
        tab_ref[...], onehot, (((0,), (0,)), ((), ())),
        preferred_element_type=jnp.float32)
    out_ref[0] = jnp.concatenate((lin, emb), axis=0)


def kernel(word, age, age_table, W, b):
    B, S, D = word.shape
    E = W.shape[1]
    A, EA = age_table.shape

    word_t = jnp.transpose(word, (1, 2, 0))
    age_t = jnp.transpose(jnp.asarray(age, jnp.int32), (1, 0))
    b_col = b.reshape(E, 1)

    CL = _LANES_PER_BLOCK
    grid = (S, B // CL)
    out_t = pl.pallas_call(
        _fused_body,
        grid=grid,
        in_specs=[
            pl.BlockSpec((1, D, CL), lambda s, j: (s, 0, j)),
            pl.BlockSpec((S, CL), lambda s, j: (0, j)),
            pl.BlockSpec((D, E), lambda s, j: (0, 0)),
            pl.BlockSpec((E, 1), lambda s, j: (0, 0)),
            pl.BlockSpec((A, EA), lambda s, j: (0, 0)),
        ],
        out_specs=pl.BlockSpec((1, E + EA, CL), lambda s, j: (s, 0, j)),
        out_shape=jax.ShapeDtypeStruct((S, E + EA, B), jnp.float32),
    )(word_t, age_t, W, b_col, age_table)
    return jnp.transpose(out_t, (2, 0, 1))
